# baseline (speedup 1.0000x reference)
import jax
import jax.numpy as jnp
from jax import lax
from jax.experimental import pallas as pl
from jax.experimental.pallas import tpu as pltpu

_N = 128
_HEADS = 4
_HID = 32
_OUT_DIM = 2
_NB = 4


def _gat_clique(h, asrc_m, adst_m, mask4, outd):
    als = jnp.dot(h, asrc_m)
    ald = jnp.dot(h, adst_m)
    lhs = jnp.concatenate([als, jnp.ones_like(als)], axis=1)
    ald_t = ald.T
    ald_tile = jnp.concatenate([ald_t] * _HEADS, axis=1)
    rhs = jnp.concatenate([mask4, ald_tile * mask4], axis=0)
    e_wide = jnp.dot(lhs, rhs)
    oh = []
    for hd in range(_HEADS):
        e = lax.slice(e_wide, (0, hd * _N), (_N, (hd + 1) * _N))
        e = jnp.maximum(e, 0.2 * e)
        m = jnp.max(e, axis=0, keepdims=True)
        ex = jnp.exp(e - m)
        den = jnp.sum(ex, axis=0, keepdims=True)
        alpha = ex / (den + 1e-16)
        h_h = lax.slice(h, (0, hd * outd), (_N, (hd + 1) * outd))
        oh.append(lax.dot_general(alpha, h_h, (((0,), (0,)), ((), ()))))
    return jnp.concatenate(oh, axis=1)


def _block_kernel(xs_ref, pe_ref, mask_ref, w1_ref, as1_ref, ad1_ref, b1_ref,
                  w2_ref, as2_ref, ad2_ref, b2_ref,
                  w3_ref, as3_ref, ad3_ref, b3_ref, out_ref):
    mask4 = mask_ref[...]
    w1 = w1_ref[...]
    pew = jnp.dot(pe_ref[0], w1)
    w1r0 = lax.slice(w1, (0, 0), (1, _N))
    xsb = xs_ref[0, 0, 0]
    o1 = []
    for b in range(_NB):
        xr = lax.slice(xsb, (b, 0), (b + 1, _N))
        h1 = pew + lax.dot_general(xr, w1r0, (((0,), (0,)), ((), ())))
        o1.append(_gat_clique(h1, as1_ref[...], ad1_ref[...], mask4, _HID))
    o1 = jnp.concatenate(o1, axis=0) + b1_ref[...]
    h2 = jnp.dot(o1, w2_ref[...])
    o2 = []
    for b in range(_NB):
        h2b = lax.slice(h2, (b * _N, 0), ((b + 1) * _N, _HEADS * _HID))
        o2.append(_gat_clique(h2b, as2_ref[...], ad2_ref[...], mask4, _HID))
    o2 = jnp.concatenate(o2, axis=0) + b2_ref[...]
    h3 = jnp.dot(o2, w3_ref[...])
    for b in range(_NB):
        h3b = lax.slice(h3, (b * _N, 0), ((b + 1) * _N, _HEADS * _OUT_DIM))
        o3b = _gat_clique(h3b, as3_ref[...], ad3_ref[...], mask4, _OUT_DIM)
        o3b = o3b + b3_ref[...]
        out_ref[b, 0, :] = jnp.mean(o3b, axis=0)


def _attn_mat(a):
    heads, outd = a.shape
    return (jnp.eye(heads, dtype=a.dtype)[:, :, None] * a[None, :, :]).reshape(
        heads, heads * outd).T


def kernel(xs, pos_enc, W1, a_src1, a_dst1, b1, W2, a_src2, a_dst2, b2,
           W3, a_src3, a_dst3, b3):
    bs, nr, nc = xs.shape
    enc = pos_enc.shape[-1]
    nblocks = bs * nr
    steps_per_graph = nr // _NB
    pe_pad = jnp.pad(pos_enc, ((0, 0), (0, 0), (1, 0)))
    xs4 = xs.reshape(bs, steps_per_graph, 1, _NB, nc)
    mask4 = jnp.repeat(jnp.eye(_HEADS, dtype=jnp.float32), _N, axis=1)

    def whole(shape):
        return pl.BlockSpec(shape, lambda i: tuple(0 for _ in shape))

    hh = _HEADS * _HID
    ho = _HEADS * _OUT_DIM
    out = pl.pallas_call(
        _block_kernel,
        grid=(nblocks // _NB,),
        in_specs=[
            pl.BlockSpec((1, 1, 1, _NB, nc),
                         lambda i: (i // steps_per_graph, i % steps_per_graph,
                                    0, 0, 0)),
            pl.BlockSpec((1, nc, 1 + enc),
                         lambda i: (i // steps_per_graph, 0, 0)),
            whole((_HEADS, _HEADS * _N)),
            whole((1 + enc, hh)),
            whole((hh, _HEADS)), whole((hh, _HEADS)), whole((1, hh)),
            whole((hh, hh)),
            whole((hh, _HEADS)), whole((hh, _HEADS)), whole((1, hh)),
            whole((hh, ho)),
            whole((ho, _HEADS)), whole((ho, _HEADS)), whole((1, ho)),
        ],
        out_specs=pl.BlockSpec((_NB, 1, ho), lambda i: (i, 0, 0)),
        out_shape=jax.ShapeDtypeStruct((nblocks, 1, ho), jnp.float32),
        compiler_params=pltpu.CompilerParams(
            dimension_semantics=("parallel",)),
    )(
        xs4, pe_pad, mask4,
        W1, _attn_mat(a_src1), _attn_mat(a_dst1), b1.reshape(1, -1),
        W2, _attn_mat(a_src2), _attn_mat(a_dst2), b2.reshape(1, -1),
        W3, _attn_mat(a_src3), _attn_mat(a_dst3), b3.reshape(1, -1),
    )
    return out.reshape(bs, nr, _HEADS * _OUT_DIM)
